# baseline (speedup 1.0000x reference)
import functools

import jax
import jax.numpy as jnp
from jax import lax
from jax.experimental import pallas as pl
from jax.experimental.pallas import tpu as pltpu
from jax.experimental.pallas import tpu_sc as plsc


def _make_sc_gather_t(batch, rows, n):
    info = plsc.get_sparse_core_info()
    nc, ns = info.num_cores, info.num_subcores
    nw = nc * ns
    b_per_w = batch // nw
    groups = b_per_w // 16
    mesh = plsc.VectorSubcoreMesh(core_axis_name="c", subcore_axis_name="s")

    @functools.partial(
        pl.kernel,
        mesh=mesh,
        out_type=jax.ShapeDtypeStruct((nw, rows, b_per_w), jnp.float32),
        scratch_types=[
            pltpu.VMEM((b_per_w,), jnp.int32),
            pltpu.VMEM((rows * 16,), jnp.float32),
            pltpu.VMEM((rows, b_per_w), jnp.float32),
        ],
    )
    def gather_k(table_hbm, idx_hbm, out_hbm, idx_v, tab_v, out_v):
        wid = lax.axis_index("s") * nc + lax.axis_index("c")
        base = wid * b_per_w
        pltpu.sync_copy(idx_hbm.at[pl.ds(base, b_per_w)], idx_v)
        pltpu.sync_copy(table_hbm, tab_v)
        masks = []
        for g in range(groups):
            idxg = idx_v[pl.ds(g * 16, 16)]
            masks.append([idxg == j for j in range(n - 1)])

        def body(r, carry):
            v = tab_v[pl.ds(r * 16, 16)]
            vals_n = [v[j] for j in range(n)]
            for g in range(groups):
                sel = jnp.full((16,), vals_n[n - 1], jnp.float32)
                for j in range(n - 2, -1, -1):
                    sel = jnp.where(masks[g][j], vals_n[j], sel)
                out_v[r, pl.ds(g * 16, 16)] = sel
            return carry

        lax.fori_loop(0, rows, body, 0)
        pltpu.sync_copy(out_v, out_hbm.at[wid])

    return gather_k


def _film_body(gb_ref, f_ref, o_ref, *, d):
    g = gb_ref[0, :d, :]
    b = gb_ref[0, d:, :]
    o_ref[...] = f_ref[...] * g[None] + b[None]


def kernel(feat, alt_idx, gamma, beta):
    batch, seq, d = feat.shape
    n = gamma.shape[0]
    idx = alt_idx.astype(jnp.int32)
    rows = 2 * d
    tab = jnp.concatenate([gamma.T, beta.T], axis=0)
    table_flat = jnp.pad(tab, ((0, 0), (0, 16 - n))).reshape(rows * 16)
    gbt = _make_sc_gather_t(batch, rows, n)(table_flat, idx)

    nw, _, bb = gbt.shape
    lblk = 50
    feat_t = feat.transpose(1, 2, 0)
    film = pl.pallas_call(
        functools.partial(_film_body, d=d),
        grid=(batch // bb, seq // lblk),
        in_specs=[
            pl.BlockSpec((1, rows, bb), lambda j, l: (j, 0, 0)),
            pl.BlockSpec((lblk, d, bb), lambda j, l: (l, 0, j)),
        ],
        out_specs=pl.BlockSpec((lblk, d, bb), lambda j, l: (l, 0, j)),
        out_shape=jax.ShapeDtypeStruct((seq, d, batch), jnp.float32),
        compiler_params=pltpu.CompilerParams(
            dimension_semantics=("parallel", "parallel"),
        ),
    )
    out_t = film(gbt, feat_t)
    return out_t.transpose(2, 0, 1)


# speedup vs baseline: 1.1757x; 1.1757x over previous
import functools

import jax
import jax.numpy as jnp
from jax import lax
from jax.experimental import pallas as pl
from jax.experimental.pallas import tpu as pltpu
from jax.experimental.pallas import tpu_sc as plsc


def _make_sc_gather_t(batch, d, n):
    info = plsc.get_sparse_core_info()
    nc, ns = info.num_cores, info.num_subcores
    nw = nc * ns
    b_per_w = batch // nw
    groups = b_per_w // 16
    rows = 2 * d
    mesh = plsc.VectorSubcoreMesh(core_axis_name="c", subcore_axis_name="s")

    @functools.partial(
        pl.kernel,
        mesh=mesh,
        out_type=jax.ShapeDtypeStruct((nw, rows, b_per_w), jnp.float32),
        scratch_types=[
            pltpu.VMEM((b_per_w,), jnp.int32),
            pltpu.VMEM((2, n, d), jnp.float32),
            pltpu.VMEM((rows, b_per_w), jnp.float32),
        ],
    )
    def gather_k(gamma_hbm, beta_hbm, idx_hbm, out_hbm, idx_v, gb_v, out_v):
        wid = lax.axis_index("s") * nc + lax.axis_index("c")
        base = wid * b_per_w
        pltpu.sync_copy(idx_hbm.at[pl.ds(base, b_per_w)], idx_v)
        pltpu.sync_copy(gamma_hbm, gb_v.at[0])
        pltpu.sync_copy(beta_hbm, gb_v.at[1])
        masks = []
        for g in range(groups):
            idxg = idx_v[pl.ds(g * 16, 16)]
            masks.append([idxg == j for j in range(n - 1)])

        def body(db, carry):
            for t in range(2):
                vj = [gb_v[t, j, pl.ds(db * 16, 16)] for j in range(n)]
                for i in range(16):
                    vals = [vj[j][i] for j in range(n)]
                    r = db * 16 + i + t * d
                    for g in range(groups):
                        sel = jnp.full((16,), vals[n - 1], jnp.float32)
                        for j in range(n - 2, -1, -1):
                            sel = jnp.where(masks[g][j], vals[j], sel)
                        out_v[r, pl.ds(g * 16, 16)] = sel
            return carry

        lax.fori_loop(0, d // 16, body, 0)
        pltpu.sync_copy(out_v, out_hbm.at[wid])

    return gather_k


def _film_body(gb_ref, f_ref, o_ref, *, d):
    g = gb_ref[0, :d, :]
    b = gb_ref[0, d:, :]
    o_ref[...] = f_ref[...] * g[None] + b[None]


def kernel(feat, alt_idx, gamma, beta):
    batch, seq, d = feat.shape
    n = gamma.shape[0]
    idx = alt_idx.astype(jnp.int32)
    rows = 2 * d
    gbt = _make_sc_gather_t(batch, d, n)(gamma, beta, idx)

    nw, _, bb = gbt.shape
    lblk = seq
    feat_t = feat.transpose(1, 2, 0)
    film = pl.pallas_call(
        functools.partial(_film_body, d=d),
        grid=(batch // bb, seq // lblk),
        in_specs=[
            pl.BlockSpec((1, rows, bb), lambda j, l: (j, 0, 0)),
            pl.BlockSpec((lblk, d, bb), lambda j, l: (l, 0, j)),
        ],
        out_specs=pl.BlockSpec((lblk, d, bb), lambda j, l: (l, 0, j)),
        out_shape=jax.ShapeDtypeStruct((seq, d, batch), jnp.float32),
        compiler_params=pltpu.CompilerParams(
            dimension_semantics=("parallel", "parallel"),
        ),
    )
    out_t = film(gbt, feat_t)
    return out_t.transpose(2, 0, 1)


# speedup vs baseline: 1.1832x; 1.0063x over previous
import functools

import jax
import jax.numpy as jnp
from jax import lax
from jax.experimental import pallas as pl
from jax.experimental.pallas import tpu as pltpu
from jax.experimental.pallas import tpu_sc as plsc


def _make_sc_gather_t(batch, d, n):
    info = plsc.get_sparse_core_info()
    nc, ns = info.num_cores, info.num_subcores
    nw = nc * ns
    b_per_w = batch // nw
    groups = b_per_w // 16
    rows = 2 * d
    mesh = plsc.VectorSubcoreMesh(core_axis_name="c", subcore_axis_name="s")

    @functools.partial(
        pl.kernel,
        mesh=mesh,
        out_type=jax.ShapeDtypeStruct((nw, rows, b_per_w), jnp.float32),
        scratch_types=[
            pltpu.VMEM((b_per_w,), jnp.int32),
            pltpu.VMEM((2, n, d), jnp.float32),
            pltpu.VMEM((rows, b_per_w), jnp.float32),
            pltpu.SemaphoreType.DMA,
            pltpu.SemaphoreType.DMA,
            pltpu.SemaphoreType.DMA,
        ],
    )
    def gather_k(gamma_hbm, beta_hbm, idx_hbm, out_hbm,
                 idx_v, gb_v, out_v, sem_i, sem_g, sem_b):
        wid = lax.axis_index("s") * nc + lax.axis_index("c")
        base = wid * b_per_w
        cp_i = pltpu.async_copy(idx_hbm.at[pl.ds(base, b_per_w)], idx_v, sem_i)
        cp_g = pltpu.async_copy(gamma_hbm, gb_v.at[0], sem_g)
        cp_b = pltpu.async_copy(beta_hbm, gb_v.at[1], sem_b)
        cp_i.wait()
        cp_g.wait()
        cp_b.wait()
        masks = []
        for g in range(groups):
            idxg = idx_v[pl.ds(g * 16, 16)]
            if n == 4:
                masks.append((idxg == 0, idxg == 2, idxg < 2))
            else:
                masks.append([idxg == j for j in range(n - 1)])

        def body(db, carry):
            for t in range(2):
                vj = [gb_v[t, j, pl.ds(db * 16, 16)] for j in range(n)]
                for i in range(16):
                    vals = [vj[j][i] for j in range(n)]
                    r = db * 16 + i + t * d
                    for g in range(groups):
                        if n == 4:
                            m0, m2, mlow = masks[g]
                            sel = jnp.where(
                                mlow,
                                jnp.where(m0, vals[0], vals[1]),
                                jnp.where(m2, vals[2], vals[3]),
                            )
                        else:
                            sel = jnp.full((16,), vals[n - 1], jnp.float32)
                            for j in range(n - 2, -1, -1):
                                sel = jnp.where(masks[g][j], vals[j], sel)
                        out_v[r, pl.ds(g * 16, 16)] = sel
            return carry

        lax.fori_loop(0, d // 16, body, 0)
        pltpu.sync_copy(out_v, out_hbm.at[wid])

    return gather_k


def _film_body(gb_ref, f_ref, o_ref, *, d):
    g = gb_ref[0, :d, :]
    b = gb_ref[0, d:, :]
    o_ref[...] = f_ref[...] * g[None] + b[None]


def kernel(feat, alt_idx, gamma, beta):
    batch, seq, d = feat.shape
    n = gamma.shape[0]
    idx = alt_idx.astype(jnp.int32)
    rows = 2 * d
    gbt = _make_sc_gather_t(batch, d, n)(gamma, beta, idx)

    nw, _, bb = gbt.shape
    lblk = seq
    feat_t = feat.transpose(1, 2, 0)
    film = pl.pallas_call(
        functools.partial(_film_body, d=d),
        grid=(batch // bb, seq // lblk),
        in_specs=[
            pl.BlockSpec((1, rows, bb), lambda j, l: (j, 0, 0)),
            pl.BlockSpec((lblk, d, bb), lambda j, l: (l, 0, j)),
        ],
        out_specs=pl.BlockSpec((lblk, d, bb), lambda j, l: (l, 0, j)),
        out_shape=jax.ShapeDtypeStruct((seq, d, batch), jnp.float32),
        compiler_params=pltpu.CompilerParams(
            dimension_semantics=("parallel", "parallel"),
        ),
    )
    out_t = film(gbt, feat_t)
    return out_t.transpose(2, 0, 1)
